# baseline (speedup 1.0000x reference)
import dataclasses
import functools

import jax
import jax.numpy as jnp
from jax import lax
from jax.experimental import pallas as pl
from jax.experimental.pallas import tpu as pltpu
from jax.experimental.pallas import tpu_sc as plsc

B = 8
SEQ = 4096
DIM = 64
NH = 8
NBKT = 64
NROT = NBKT // 2
CH = 64
TOT = NH * SEQ
GTOT = B * TOT
CB = 256
NCB = SEQ // CB
NB = 4
KW = (NB + 1) * CH
M = NB * CH
CPB = TOT // CH
NCHUNK = GTOT // CH
PACK = 128
W = 128
NW = GTOT // W
NWORK = 32
SELF_MASK_VAL = -50000.0
NEG_BIG = -1e30


def _hash_sort_body(qk_ref, rot_ref, slot_ref):
    bh = pl.program_id(0)
    qk = qk_ref[0]
    rot = rot_ref[0]
    rvT = lax.dot_general(rot.astype(jnp.bfloat16), qk.astype(jnp.bfloat16),
                          (((0,), (1,)), ((), ())),
                          preferred_element_type=jnp.float32)
    m1 = jnp.max(rvT, axis=0, keepdims=True)
    m2 = jnp.max(-rvT, axis=0, keepdims=True)
    sub = lax.broadcasted_iota(jnp.int32, (NROT, SEQ), 0)
    idx1 = jnp.min(jnp.where(rvT == m1, sub, NBKT), axis=0, keepdims=True)
    idx2 = jnp.min(jnp.where(-rvT == m2, sub, NBKT), axis=0, keepdims=True)
    bucket = jnp.where(m1 >= m2, idx1, idx2 + NROT)

    bsub = lax.broadcasted_iota(jnp.int32, (NBKT, SEQ), 0)
    onehot = bsub == bucket
    oh_f = onehot.astype(jnp.float32)
    oh_b = onehot.astype(jnp.bfloat16)

    hist = jnp.sum(oh_f, axis=1, keepdims=True)
    inc = hist
    for sh in (1, 2, 4, 8, 16, 32):
        inc = inc + jnp.concatenate(
            [jnp.zeros((sh, 1), jnp.float32), inc[:-sh, :]], axis=0)
    start = inc - hist

    ri = lax.broadcasted_iota(jnp.int32, (CB, CB), 0)
    ci = lax.broadcasted_iota(jnp.int32, (CB, CB), 1)
    utri = (ri < ci).astype(jnp.bfloat16)

    base = jnp.zeros((NBKT, 1), jnp.float32)
    for k in range(NCB):
        ohk_b = oh_b[:, k * CB:(k + 1) * CB]
        ohk_f = oh_f[:, k * CB:(k + 1) * CB]
        cumk = lax.dot_general(ohk_b, utri, (((1,), (0,)), ((), ())),
                               preferred_element_type=jnp.float32)
        pos = jnp.sum(ohk_f * (cumk + base + start), axis=0, keepdims=True)
        slot_ref[0, :, k * CB:(k + 1) * CB] = pos.astype(jnp.int32) + bh * SEQ
        base = base + jnp.sum(ohk_f, axis=1, keepdims=True)


def _hash_sort(qk, rot_h):
    return pl.pallas_call(
        _hash_sort_body,
        grid=(B * NH,),
        in_specs=[
            pl.BlockSpec((1, SEQ, DIM), lambda bh: (bh // NH, 0, 0)),
            pl.BlockSpec((1, DIM, NROT), lambda bh: (bh % NH, 0, 0)),
        ],
        out_specs=pl.BlockSpec((1, 1, SEQ), lambda bh: (bh, 0, 0)),
        out_shape=jax.ShapeDtypeStruct((B * NH, 1, SEQ), jnp.int32),
    )(qk, rot_h)


@functools.lru_cache(maxsize=None)
def _sc_mesh():
    return plsc.VectorSubcoreMesh(core_axis_name="c", subcore_axis_name="s")


def _sc_compiler_params():
    cp = pltpu.CompilerParams()
    if "needs_layout_passes" in pltpu.CompilerParams.__dataclass_fields__:
        cp = dataclasses.replace(cp, needs_layout_passes=False)
    return cp


@functools.lru_cache(maxsize=None)
def _sc_scatter():
    @functools.partial(
        pl.kernel,
        compiler_params=_sc_compiler_params(),
        out_type=[
            jax.ShapeDtypeStruct((GTOT, 2 * DIM), jnp.float32),
            jax.ShapeDtypeStruct((GTOT,), jnp.int32),
        ],
        mesh=_sc_mesh(),
        scratch_types=[
            pltpu.VMEM((W,), jnp.int32),
            pltpu.VMEM((W, 2 * DIM), jnp.float32),
            pltpu.VMEM((SEQ,), jnp.int32),
            pltpu.VMEM((SEQ,), jnp.int32),
        ],
    )
    def body(qkv_hbm, slot_hbm, sqkv_hbm, st_hbm, idx_v, row_v, slot_v, st_v):
        wid = lax.axis_index("s") * 2 + lax.axis_index("c")
        wpb = TOT // W
        wpr = SEQ // W

        @pl.loop(0, NW // NWORK)
        def _(j):
            w = wid + j * NWORK
            bb = w // wpb
            t0 = (w % wpr) * W
            row0 = bb * SEQ + t0
            pltpu.sync_copy(slot_hbm.at[pl.ds(w * W, W)], idx_v)
            pltpu.sync_copy(qkv_hbm.at[pl.ds(row0, W)], row_v)
            pltpu.sync_copy(row_v, sqkv_hbm.at[idx_v])

        @pl.loop(0, B * NH // NWORK)
        def _(r):
            bh = wid + r * NWORK
            pltpu.sync_copy(slot_hbm.at[pl.ds(bh * SEQ, SEQ)], slot_v)
            base = bh * SEQ

            @pl.loop(0, SEQ // 16)
            def _(k):
                sl = pl.ds(k * 16, 16)
                loc = slot_v[sl] - base
                tvec = lax.iota(jnp.int32, 16) + k * 16
                plsc.store_scatter(st_v, [loc], tvec)

            pltpu.sync_copy(st_v, st_hbm.at[pl.ds(base, SEQ)])

    return body


def _attn_body(qv_ref, qvh_ref, stq_ref, stk_ref, sth_ref, out_ref):
    qv = qv_ref[...].reshape(M, 2 * DIM)
    qvh = qvh_ref[...].reshape(CH, 2 * DIM)
    q = qv[:, :DIM]
    kq = jnp.concatenate([qvh[:, :DIM], q], axis=0)
    vv = jnp.concatenate([qvh[:, DIM:], qv[:, DIM:]], axis=0)
    qt = stq_ref[...].reshape(M, 1)
    stk = stk_ref[...]

    nrm = jnp.sqrt(jnp.sum(kq * kq, axis=1, keepdims=True))
    bk = (kq / jnp.maximum(nrm, 1e-12)).astype(jnp.bfloat16)

    dots = lax.dot_general(q.astype(jnp.bfloat16), bk,
                           (((1,), (1,)), ((), ())),
                           preferred_element_type=jnp.float32)
    dots = dots * (DIM ** -0.5)
    pieces = []
    for kc in range(NB + 1):
        krow = (sth_ref[...] if kc == 0 else stk[kc - 1:kc]).reshape(1, CH)
        pieces.append(jnp.where(qt == krow, SELF_MASK_VAL,
                                dots[:, kc * CH:(kc + 1) * CH]))
    dots = jnp.concatenate(pieces, axis=1)
    ri = lax.broadcasted_iota(jnp.int32, (M, KW), 0)
    ci = lax.broadcasted_iota(jnp.int32, (M, KW), 1)
    band = ci // CH - ri // CH
    dots = jnp.where((band >= 0) & (band <= 1), dots, NEG_BIG)

    mx = jnp.max(dots, axis=1, keepdims=True)
    p = jnp.exp(dots - mx)
    s = jnp.sum(p, axis=1, keepdims=True)
    lse = mx + jnp.log(s)
    probs = (p / s).astype(jnp.bfloat16)
    bo = lax.dot_general(probs, vv.astype(jnp.bfloat16),
                         (((1,), (0,)), ((), ())),
                         preferred_element_type=jnp.float32)
    out = jnp.concatenate(
        [bo, lse, jnp.zeros((M, PACK - DIM - 1), jnp.float32)], axis=1)
    out_ref[...] = out.reshape(NB, CH, PACK)


def _halo_idx(c):
    g0 = c * NB
    return jnp.where(g0 % CPB == 0, g0 + CPB - 1, g0 - 1)


def _attention(sqkv3, stq3, stk3):
    return pl.pallas_call(
        _attn_body,
        grid=(NCHUNK // NB,),
        in_specs=[
            pl.BlockSpec((NB, CH, 2 * DIM), lambda c: (c, 0, 0)),
            pl.BlockSpec((1, CH, 2 * DIM), lambda c: (_halo_idx(c), 0, 0)),
            pl.BlockSpec((NB, CH, 1), lambda c: (c, 0, 0)),
            pl.BlockSpec((NB, 1, CH), lambda c: (c, 0, 0)),
            pl.BlockSpec((1, 1, CH), lambda c: (_halo_idx(c), 0, 0)),
        ],
        out_specs=pl.BlockSpec((NB, CH, PACK), lambda c: (c, 0, 0)),
        out_shape=jax.ShapeDtypeStruct((NCHUNK, CH, PACK), jnp.float32),
    )(sqkv3, sqkv3, stq3, stk3, stk3)


@functools.lru_cache(maxsize=None)
def _sc_unsort():
    @functools.partial(
        pl.kernel,
        out_type=jax.ShapeDtypeStruct((GTOT, PACK), jnp.float32),
        mesh=_sc_mesh(),
        scratch_types=[
            pltpu.VMEM((W,), jnp.int32),
            pltpu.VMEM((W, PACK), jnp.float32),
            pltpu.SemaphoreType.DMA,
        ],
    )
    def body(so_hbm, slot_hbm, out_hbm, idx_v, rows_v, sem):
        wid = lax.axis_index("s") * 2 + lax.axis_index("c")

        @pl.loop(0, NW // NWORK)
        def _(j):
            w = wid + j * NWORK
            pltpu.sync_copy(slot_hbm.at[pl.ds(w * W, W)], idx_v)
            pltpu.async_copy(so_hbm.at[idx_v], rows_v, sem).wait()
            pltpu.sync_copy(rows_v, out_hbm.at[pl.ds(w * W, W)])

    return body


TB = 512


def _combine_body(g_ref, out_ref):
    g = g_ref[0]
    o = g[:, :, :DIM]
    lg = g[:, :, DIM:DIM + 1]
    mx = jnp.max(lg, axis=0, keepdims=True)
    wgt = jnp.exp(lg - mx)
    s = jnp.sum(wgt, axis=0, keepdims=True)
    wn = wgt / s
    out_ref[0] = jnp.sum(o * wn, axis=0)


def _combine(g4):
    return pl.pallas_call(
        _combine_body,
        grid=(B, SEQ // TB),
        in_specs=[pl.BlockSpec((1, NH, TB, PACK), lambda b, t: (b, 0, t, 0))],
        out_specs=pl.BlockSpec((1, TB, DIM), lambda b, t: (b, t, 0)),
        out_shape=jax.ShapeDtypeStruct((B, SEQ, DIM), jnp.float32),
    )(g4)


def kernel(qk, v, rotations):
    rot_h = jnp.transpose(rotations[0], (1, 0, 2))
    slot = _hash_sort(qk, rot_h).reshape(GTOT)

    qkv = jnp.concatenate(
        [qk.reshape(B * SEQ, DIM), v.reshape(B * SEQ, DIM)], axis=1)
    sqkv, st = _sc_scatter()(qkv, slot)

    so = _attention(sqkv.reshape(NCHUNK, CH, 2 * DIM),
                    st.reshape(NCHUNK, CH, 1),
                    st.reshape(NCHUNK, 1, CH))

    g = _sc_unsort()(so.reshape(GTOT, PACK), slot)
    return _combine(g.reshape(B, NH, SEQ, PACK))


# speedup vs baseline: 1.1082x; 1.1082x over previous
import dataclasses
import functools

import jax
import jax.numpy as jnp
from jax import lax
from jax.experimental import pallas as pl
from jax.experimental.pallas import tpu as pltpu
from jax.experimental.pallas import tpu_sc as plsc

B = 8
SEQ = 4096
DIM = 64
NH = 8
NBKT = 64
NROT = NBKT // 2
CH = 64
TOT = NH * SEQ
GTOT = B * TOT
CB = 256
NCB = SEQ // CB
NB = 4
KW = (NB + 1) * CH
M = NB * CH
CPB = TOT // CH
PACK = 128
W = 128
NWORK = 32

NSPLIT = 2
HB = B // NSPLIT
GTOT_S = HB * TOT
NW_S = GTOT_S // W
NCHUNK_S = GTOT_S // CH
NRND_S = HB * NH
SELF_MASK_VAL = -50000.0
NEG_BIG = -1e30


def _hash_sort_body(qk_ref, rot_ref, slot_ref):
    bh = pl.program_id(0)
    qk = qk_ref[0]
    rot = rot_ref[0]
    rvT = lax.dot_general(rot.astype(jnp.bfloat16), qk.astype(jnp.bfloat16),
                          (((0,), (1,)), ((), ())),
                          preferred_element_type=jnp.float32)
    m1 = jnp.max(rvT, axis=0, keepdims=True)
    m2 = jnp.max(-rvT, axis=0, keepdims=True)
    sub = lax.broadcasted_iota(jnp.int32, (NROT, SEQ), 0)
    idx1 = jnp.min(jnp.where(rvT == m1, sub, NBKT), axis=0, keepdims=True)
    idx2 = jnp.min(jnp.where(-rvT == m2, sub, NBKT), axis=0, keepdims=True)
    bucket = jnp.where(m1 >= m2, idx1, idx2 + NROT)

    bsub = lax.broadcasted_iota(jnp.int32, (NBKT, SEQ), 0)
    onehot = bsub == bucket
    oh_f = onehot.astype(jnp.float32)
    oh_b = onehot.astype(jnp.bfloat16)

    hist = jnp.sum(oh_f, axis=1, keepdims=True)
    inc = hist
    for sh in (1, 2, 4, 8, 16, 32):
        inc = inc + jnp.concatenate(
            [jnp.zeros((sh, 1), jnp.float32), inc[:-sh, :]], axis=0)
    start = inc - hist

    ri = lax.broadcasted_iota(jnp.int32, (CB, CB), 0)
    ci = lax.broadcasted_iota(jnp.int32, (CB, CB), 1)
    utri = (ri < ci).astype(jnp.bfloat16)

    base = jnp.zeros((NBKT, 1), jnp.float32)
    for k in range(NCB):
        ohk_b = oh_b[:, k * CB:(k + 1) * CB]
        ohk_f = oh_f[:, k * CB:(k + 1) * CB]
        cumk = lax.dot_general(ohk_b, utri, (((1,), (0,)), ((), ())),
                               preferred_element_type=jnp.float32)
        pos = jnp.sum(ohk_f * (cumk + base + start), axis=0, keepdims=True)
        slot_ref[0, :, k * CB:(k + 1) * CB] = (
            pos.astype(jnp.int32) + (bh % NRND_S) * SEQ)
        base = base + jnp.sum(ohk_f, axis=1, keepdims=True)


def _hash_sort(qk, rot_h):
    return pl.pallas_call(
        _hash_sort_body,
        grid=(B * NH,),
        in_specs=[
            pl.BlockSpec((1, SEQ, DIM), lambda bh: (bh // NH, 0, 0)),
            pl.BlockSpec((1, DIM, NROT), lambda bh: (bh % NH, 0, 0)),
        ],
        out_specs=pl.BlockSpec((1, 1, SEQ), lambda bh: (bh, 0, 0)),
        out_shape=jax.ShapeDtypeStruct((B * NH, 1, SEQ), jnp.int32),
    )(qk, rot_h)


@functools.lru_cache(maxsize=None)
def _sc_mesh():
    return plsc.VectorSubcoreMesh(core_axis_name="c", subcore_axis_name="s")


def _sc_compiler_params():
    cp = pltpu.CompilerParams()
    if "needs_layout_passes" in pltpu.CompilerParams.__dataclass_fields__:
        cp = dataclasses.replace(cp, needs_layout_passes=False)
    return cp


@functools.lru_cache(maxsize=None)
def _sc_scatter():
    @functools.partial(
        pl.kernel,
        compiler_params=_sc_compiler_params(),
        out_type=[
            jax.ShapeDtypeStruct((GTOT_S, 2 * DIM), jnp.float32),
            jax.ShapeDtypeStruct((GTOT_S,), jnp.int32),
        ],
        mesh=_sc_mesh(),
        scratch_types=[
            pltpu.VMEM((W,), jnp.int32),
            pltpu.VMEM((W, 2 * DIM), jnp.float32),
            pltpu.VMEM((SEQ,), jnp.int32),
            pltpu.VMEM((SEQ,), jnp.int32),
        ],
    )
    def body(qkv_hbm, slot_hbm, sqkv_hbm, st_hbm, idx_v, row_v, slot_v, st_v):
        wid = lax.axis_index("s") * 2 + lax.axis_index("c")
        wpb = TOT // W
        wpr = SEQ // W

        @pl.loop(0, NW_S // NWORK)
        def _(j):
            w = wid + j * NWORK
            bb = w // wpb
            t0 = (w % wpr) * W
            row0 = bb * SEQ + t0
            pltpu.sync_copy(slot_hbm.at[pl.ds(w * W, W)], idx_v)
            pltpu.sync_copy(qkv_hbm.at[pl.ds(row0, W)], row_v)
            pltpu.sync_copy(row_v, sqkv_hbm.at[idx_v])

        @pl.loop(0, NRND_S // NWORK)
        def _(r):
            bh = wid + r * NWORK
            pltpu.sync_copy(slot_hbm.at[pl.ds(bh * SEQ, SEQ)], slot_v)
            base = bh * SEQ

            @pl.loop(0, SEQ // 16)
            def _(k):
                sl = pl.ds(k * 16, 16)
                loc = slot_v[sl] - base
                tvec = lax.iota(jnp.int32, 16) + k * 16
                plsc.store_scatter(st_v, [loc], tvec)

            pltpu.sync_copy(st_v, st_hbm.at[pl.ds(base, SEQ)])

    return body


def _attn_body(qv_ref, qvh_ref, stq_ref, stk_ref, sth_ref, out_ref):
    qv = qv_ref[...].reshape(M, 2 * DIM)
    qvh = qvh_ref[...].reshape(CH, 2 * DIM)
    q = qv[:, :DIM]
    kq = jnp.concatenate([qvh[:, :DIM], q], axis=0)
    vv = jnp.concatenate([qvh[:, DIM:], qv[:, DIM:]], axis=0)
    qt = stq_ref[...].reshape(M, 1)
    stk = stk_ref[...]

    nrm = jnp.sqrt(jnp.sum(kq * kq, axis=1, keepdims=True))
    bk = (kq / jnp.maximum(nrm, 1e-12)).astype(jnp.bfloat16)

    dots = lax.dot_general(q.astype(jnp.bfloat16), bk,
                           (((1,), (1,)), ((), ())),
                           preferred_element_type=jnp.float32)
    dots = dots * (DIM ** -0.5)
    rowi = lax.broadcasted_iota(jnp.int32, (M, 1), 0)
    pieces = []
    for kc in range(NB + 1):
        krow = (sth_ref[...] if kc == 0 else stk[kc - 1:kc]).reshape(1, CH)
        d_c = jnp.where(qt == krow, SELF_MASK_VAL,
                        dots[:, kc * CH:(kc + 1) * CH])
        inband = (rowi >= (kc - 1) * CH) & (rowi < (kc + 1) * CH)
        pieces.append(jnp.where(inband, d_c, NEG_BIG))
    dots = jnp.concatenate(pieces, axis=1)

    mx = jnp.max(dots, axis=1, keepdims=True)
    p = jnp.exp(dots - mx)
    s = jnp.sum(p, axis=1, keepdims=True)
    lse = mx + jnp.log(s)
    probs = (p / s).astype(jnp.bfloat16)
    bo = lax.dot_general(probs, vv.astype(jnp.bfloat16),
                         (((1,), (0,)), ((), ())),
                         preferred_element_type=jnp.float32)
    out = jnp.concatenate(
        [bo, lse, jnp.zeros((M, PACK - DIM - 1), jnp.float32)], axis=1)
    out_ref[...] = out.reshape(NB, CH, PACK)


def _halo_idx(c):
    g0 = c * NB
    return jnp.where(g0 % CPB == 0, g0 + CPB - 1, g0 - 1)


def _attention(sqkv3, stq3, stk3):
    return pl.pallas_call(
        _attn_body,
        grid=(NCHUNK_S // NB,),
        in_specs=[
            pl.BlockSpec((NB, CH, 2 * DIM), lambda c: (c, 0, 0)),
            pl.BlockSpec((1, CH, 2 * DIM), lambda c: (_halo_idx(c), 0, 0)),
            pl.BlockSpec((NB, CH, 1), lambda c: (c, 0, 0)),
            pl.BlockSpec((NB, 1, CH), lambda c: (c, 0, 0)),
            pl.BlockSpec((1, 1, CH), lambda c: (_halo_idx(c), 0, 0)),
        ],
        out_specs=pl.BlockSpec((NB, CH, PACK), lambda c: (c, 0, 0)),
        out_shape=jax.ShapeDtypeStruct((NCHUNK_S, CH, PACK), jnp.float32),
    )(sqkv3, sqkv3, stq3, stk3, stk3)


@functools.lru_cache(maxsize=None)
def _sc_unsort():
    @functools.partial(
        pl.kernel,
        out_type=jax.ShapeDtypeStruct((GTOT_S, PACK), jnp.float32),
        mesh=_sc_mesh(),
        scratch_types=[
            pltpu.VMEM((W,), jnp.int32),
            pltpu.VMEM((W, PACK), jnp.float32),
            pltpu.SemaphoreType.DMA,
        ],
    )
    def body(so_hbm, slot_hbm, out_hbm, idx_v, rows_v, sem):
        wid = lax.axis_index("s") * 2 + lax.axis_index("c")

        @pl.loop(0, NW_S // NWORK)
        def _(j):
            w = wid + j * NWORK
            pltpu.sync_copy(slot_hbm.at[pl.ds(w * W, W)], idx_v)
            pltpu.async_copy(so_hbm.at[idx_v], rows_v, sem).wait()
            pltpu.sync_copy(rows_v, out_hbm.at[pl.ds(w * W, W)])

    return body


TB = 512


def _combine_body(g_ref, out_ref):
    g = g_ref[0]
    o = g[:, :, :DIM]
    lg = g[:, :, DIM:DIM + 1]
    mx = jnp.max(lg, axis=0, keepdims=True)
    wgt = jnp.exp(lg - mx)
    s = jnp.sum(wgt, axis=0, keepdims=True)
    wn = wgt / s
    out_ref[0] = jnp.sum(o * wn, axis=0)


def _combine(g4):
    return pl.pallas_call(
        _combine_body,
        grid=(HB, SEQ // TB),
        in_specs=[pl.BlockSpec((1, NH, TB, PACK), lambda b, t: (b, 0, t, 0))],
        out_specs=pl.BlockSpec((1, TB, DIM), lambda b, t: (b, t, 0)),
        out_shape=jax.ShapeDtypeStruct((HB, SEQ, DIM), jnp.float32),
    )(g4)


def kernel(qk, v, rotations):
    rot_h = jnp.transpose(rotations[0], (1, 0, 2))
    slot_all = _hash_sort(qk, rot_h)

    qkv = jnp.concatenate(
        [qk.reshape(B * SEQ, DIM), v.reshape(B * SEQ, DIM)], axis=1)

    outs = []
    for s in range(NSPLIT):
        slot_s = slot_all[s * NRND_S:(s + 1) * NRND_S].reshape(GTOT_S)
        qkv_s = qkv[s * HB * SEQ:(s + 1) * HB * SEQ]
        sqkv, st = _sc_scatter()(qkv_s, slot_s)
        so = _attention(sqkv.reshape(NCHUNK_S, CH, 2 * DIM),
                        st.reshape(NCHUNK_S, CH, 1),
                        st.reshape(NCHUNK_S, 1, CH))
        g = _sc_unsort()(so.reshape(GTOT_S, PACK), slot_s)
        outs.append(_combine(g.reshape(HB, NH, SEQ, PACK)))
    return jnp.concatenate(outs, axis=0)


# speedup vs baseline: 1.2801x; 1.1551x over previous
import dataclasses
import functools

import jax
import jax.numpy as jnp
from jax import lax
from jax.experimental import pallas as pl
from jax.experimental.pallas import tpu as pltpu
from jax.experimental.pallas import tpu_sc as plsc

B = 8
SEQ = 4096
DIM = 64
NH = 8
NBKT = 64
NROT = NBKT // 2
CH = 64
TOT = NH * SEQ
GTOT = B * TOT
CB = 256
NCB = SEQ // CB
NB = 4
NBG = 2 * NB
KW = (NB + 1) * CH
M = NB * CH
CPB = TOT // CH
PACK = 128
W = 128
NWORK = 32

NSPLIT = 2
HB = B // NSPLIT
GTOT_S = HB * TOT
NW_S = GTOT_S // W
NCHUNK_S = GTOT_S // CH
NRND_S = HB * NH
SELF_MASK_VAL = -50000.0
NEG_BIG = -1e30


def _hash_sort_body(qk_ref, rot_ref, slot_ref):
    bh = pl.program_id(0)
    qk = qk_ref[0]
    rot = rot_ref[0]
    rvT = lax.dot_general(rot.astype(jnp.bfloat16), qk.astype(jnp.bfloat16),
                          (((0,), (1,)), ((), ())),
                          preferred_element_type=jnp.float32)
    m1 = jnp.max(rvT, axis=0, keepdims=True)
    m2 = jnp.max(-rvT, axis=0, keepdims=True)
    sub = lax.broadcasted_iota(jnp.int32, (NROT, SEQ), 0)
    idx1 = jnp.min(jnp.where(rvT == m1, sub, NBKT), axis=0, keepdims=True)
    idx2 = jnp.min(jnp.where(-rvT == m2, sub, NBKT), axis=0, keepdims=True)
    bucket = jnp.where(m1 >= m2, idx1, idx2 + NROT)

    bsub = lax.broadcasted_iota(jnp.int32, (NBKT, SEQ), 0)
    onehot = bsub == bucket
    oh_f = onehot.astype(jnp.float32)
    oh_b = onehot.astype(jnp.bfloat16)

    hist = jnp.sum(oh_f, axis=1, keepdims=True)
    inc = hist
    for sh in (1, 2, 4, 8, 16, 32):
        inc = inc + jnp.concatenate(
            [jnp.zeros((sh, 1), jnp.float32), inc[:-sh, :]], axis=0)
    start = inc - hist

    ri = lax.broadcasted_iota(jnp.int32, (CB, CB), 0)
    ci = lax.broadcasted_iota(jnp.int32, (CB, CB), 1)
    utri = (ri < ci).astype(jnp.bfloat16)

    base = jnp.zeros((NBKT, 1), jnp.float32)
    for k in range(NCB):
        ohk_b = oh_b[:, k * CB:(k + 1) * CB]
        ohk_f = oh_f[:, k * CB:(k + 1) * CB]
        cumk = lax.dot_general(ohk_b, utri, (((1,), (0,)), ((), ())),
                               preferred_element_type=jnp.float32)
        pos = jnp.sum(ohk_f * (cumk + base + start), axis=0, keepdims=True)
        slot_ref[0, :, k * CB:(k + 1) * CB] = (
            pos.astype(jnp.int32) + (bh % NRND_S) * SEQ)
        base = base + jnp.sum(ohk_f, axis=1, keepdims=True)


def _hash_sort(qk, rot_h):
    return pl.pallas_call(
        _hash_sort_body,
        grid=(B * NH,),
        in_specs=[
            pl.BlockSpec((1, SEQ, DIM), lambda bh: (bh // NH, 0, 0)),
            pl.BlockSpec((1, DIM, NROT), lambda bh: (bh % NH, 0, 0)),
        ],
        out_specs=pl.BlockSpec((1, 1, SEQ), lambda bh: (bh, 0, 0)),
        out_shape=jax.ShapeDtypeStruct((B * NH, 1, SEQ), jnp.int32),
    )(qk, rot_h)


@functools.lru_cache(maxsize=None)
def _sc_mesh():
    return plsc.VectorSubcoreMesh(core_axis_name="c", subcore_axis_name="s")


def _sc_compiler_params():
    cp = pltpu.CompilerParams()
    if "needs_layout_passes" in pltpu.CompilerParams.__dataclass_fields__:
        cp = dataclasses.replace(cp, needs_layout_passes=False)
    return cp


@functools.lru_cache(maxsize=None)
def _sc_scatter():
    @functools.partial(
        pl.kernel,
        compiler_params=_sc_compiler_params(),
        out_type=[
            jax.ShapeDtypeStruct((GTOT_S, 2 * DIM), jnp.float32),
            jax.ShapeDtypeStruct((GTOT_S,), jnp.int32),
        ],
        mesh=_sc_mesh(),
        scratch_types=[
            pltpu.VMEM((W,), jnp.int32),
            pltpu.VMEM((W, 2 * DIM), jnp.float32),
            pltpu.VMEM((SEQ,), jnp.int32),
            pltpu.VMEM((SEQ,), jnp.int32),
        ],
    )
    def body(qkv_hbm, slot_hbm, sqkv_hbm, st_hbm, idx_v, row_v, slot_v, st_v):
        wid = lax.axis_index("s") * 2 + lax.axis_index("c")
        wpb = TOT // W
        wpr = SEQ // W

        @pl.loop(0, NW_S // NWORK)
        def _(j):
            w = wid + j * NWORK
            bb = w // wpb
            t0 = (w % wpr) * W
            row0 = bb * SEQ + t0
            pltpu.sync_copy(slot_hbm.at[pl.ds(w * W, W)], idx_v)
            pltpu.sync_copy(qkv_hbm.at[pl.ds(row0, W)], row_v)
            pltpu.sync_copy(row_v, sqkv_hbm.at[idx_v])

        @pl.loop(0, NRND_S // NWORK)
        def _(r):
            bh = wid + r * NWORK
            pltpu.sync_copy(slot_hbm.at[pl.ds(bh * SEQ, SEQ)], slot_v)
            base = bh * SEQ

            @pl.loop(0, SEQ // 16)
            def _(k):
                sl = pl.ds(k * 16, 16)
                loc = slot_v[sl] - base
                tvec = lax.iota(jnp.int32, 16) + k * 16
                plsc.store_scatter(st_v, [loc], tvec)

            pltpu.sync_copy(st_v, st_hbm.at[pl.ds(base, SEQ)])

    return body


def _attn_body(qv_ref, qvh_ref, stq_ref, stk_ref, sth_ref, out_ref):
    for h_ in range(2):
        off = h_ * NB
        qv = qv_ref[off:off + NB].reshape(M, 2 * DIM)
        if h_ == 0:
            qvh = qvh_ref[...].reshape(CH, 2 * DIM)
            sth = sth_ref[...].reshape(1, CH)
        else:
            qvh = qv_ref[off - 1]
            sth = stk_ref[off - 1]
        q = qv[:, :DIM]
        kq = jnp.concatenate([qvh[:, :DIM], q], axis=0)
        vv = jnp.concatenate([qvh[:, DIM:], qv[:, DIM:]], axis=0)
        qt = stq_ref[off:off + NB].reshape(M, 1)
        stk = stk_ref[off:off + NB]

        nrm = jnp.sqrt(jnp.sum(kq * kq, axis=1, keepdims=True))
        bk = (kq / jnp.maximum(nrm, 1e-12)).astype(jnp.bfloat16)

        dots = lax.dot_general(q.astype(jnp.bfloat16), bk,
                               (((1,), (1,)), ((), ())),
                               preferred_element_type=jnp.float32)
        dots = dots * (DIM ** -0.5)
        rowi = lax.broadcasted_iota(jnp.int32, (M, 1), 0)
        pieces = []
        for kc in range(NB + 1):
            krow = (sth if kc == 0 else stk[kc - 1]).reshape(1, CH)
            d_c = jnp.where(qt == krow, SELF_MASK_VAL,
                            dots[:, kc * CH:(kc + 1) * CH])
            inband = (rowi >= (kc - 1) * CH) & (rowi < (kc + 1) * CH)
            pieces.append(jnp.where(inband, d_c, NEG_BIG))
        dots = jnp.concatenate(pieces, axis=1)

        mx = jnp.max(dots, axis=1, keepdims=True)
        p = jnp.exp(dots - mx)
        s = jnp.sum(p, axis=1, keepdims=True)
        lse = mx + jnp.log(s)
        probs = (p / s).astype(jnp.bfloat16)
        bo = lax.dot_general(probs, vv.astype(jnp.bfloat16),
                             (((1,), (0,)), ((), ())),
                             preferred_element_type=jnp.float32)
        out = jnp.concatenate(
            [bo, lse, jnp.zeros((M, PACK - DIM - 1), jnp.float32)], axis=1)
        out_ref[off:off + NB] = out.reshape(NB, CH, PACK)


def _halo_idx(c):
    g0 = c * NBG
    return jnp.where(g0 % CPB == 0, g0 + CPB - 1, g0 - 1)


def _attention(sqkv3, stq3, stk3):
    return pl.pallas_call(
        _attn_body,
        grid=(NCHUNK_S // NBG,),
        in_specs=[
            pl.BlockSpec((NBG, CH, 2 * DIM), lambda c: (c, 0, 0)),
            pl.BlockSpec((1, CH, 2 * DIM), lambda c: (_halo_idx(c), 0, 0)),
            pl.BlockSpec((NBG, CH, 1), lambda c: (c, 0, 0)),
            pl.BlockSpec((NBG, 1, CH), lambda c: (c, 0, 0)),
            pl.BlockSpec((1, 1, CH), lambda c: (_halo_idx(c), 0, 0)),
        ],
        out_specs=pl.BlockSpec((NBG, CH, PACK), lambda c: (c, 0, 0)),
        out_shape=jax.ShapeDtypeStruct((NCHUNK_S, CH, PACK), jnp.float32),
    )(sqkv3, sqkv3, stq3, stk3, stk3)


@functools.lru_cache(maxsize=None)
def _sc_unsort():
    @functools.partial(
        pl.kernel,
        out_type=jax.ShapeDtypeStruct((GTOT_S, PACK), jnp.float32),
        mesh=_sc_mesh(),
        scratch_types=[
            pltpu.VMEM((W,), jnp.int32),
            pltpu.VMEM((W, PACK), jnp.float32),
            pltpu.SemaphoreType.DMA,
        ],
    )
    def body(so_hbm, slot_hbm, out_hbm, idx_v, rows_v, sem):
        wid = lax.axis_index("s") * 2 + lax.axis_index("c")

        @pl.loop(0, NW_S // NWORK)
        def _(j):
            w = wid + j * NWORK
            pltpu.sync_copy(slot_hbm.at[pl.ds(w * W, W)], idx_v)
            pltpu.async_copy(so_hbm.at[idx_v], rows_v, sem).wait()
            pltpu.sync_copy(rows_v, out_hbm.at[pl.ds(w * W, W)])

    return body


TB = 512


def _combine_body(g_ref, out_ref):
    g = g_ref[0]
    o = g[:, :, :DIM]
    lg = g[:, :, DIM:DIM + 1]
    mx = jnp.max(lg, axis=0, keepdims=True)
    wgt = jnp.exp(lg - mx)
    s = jnp.sum(wgt, axis=0, keepdims=True)
    wn = wgt / s
    out_ref[0] = jnp.sum(o * wn, axis=0)


def _combine(g4):
    return pl.pallas_call(
        _combine_body,
        grid=(HB, SEQ // TB),
        in_specs=[pl.BlockSpec((1, NH, TB, PACK), lambda b, t: (b, 0, t, 0))],
        out_specs=pl.BlockSpec((1, TB, DIM), lambda b, t: (b, t, 0)),
        out_shape=jax.ShapeDtypeStruct((HB, SEQ, DIM), jnp.float32),
    )(g4)


def kernel(qk, v, rotations):
    rot_h = jnp.transpose(rotations[0], (1, 0, 2))
    slot_all = _hash_sort(qk, rot_h)

    qkv = jnp.concatenate(
        [qk.reshape(B * SEQ, DIM), v.reshape(B * SEQ, DIM)], axis=1)

    outs = []
    for s in range(NSPLIT):
        slot_s = slot_all[s * NRND_S:(s + 1) * NRND_S].reshape(GTOT_S)
        qkv_s = qkv[s * HB * SEQ:(s + 1) * HB * SEQ]
        sqkv, st = _sc_scatter()(qkv_s, slot_s)
        so = _attention(sqkv.reshape(NCHUNK_S, CH, 2 * DIM),
                        st.reshape(NCHUNK_S, CH, 1),
                        st.reshape(NCHUNK_S, 1, CH))
        g = _sc_unsort()(so.reshape(GTOT_S, PACK), slot_s)
        outs.append(_combine(g.reshape(HB, NH, SEQ, PACK)))
    return jnp.concatenate(outs, axis=0)


# speedup vs baseline: 1.3156x; 1.0278x over previous
import dataclasses
import functools

import jax
import jax.numpy as jnp
from jax import lax
from jax.experimental import pallas as pl
from jax.experimental.pallas import tpu as pltpu
from jax.experimental.pallas import tpu_sc as plsc

B = 8
SEQ = 4096
DIM = 64
NH = 8
NBKT = 64
NROT = NBKT // 2
CH = 64
TOT = NH * SEQ
GTOT = B * TOT
CB = 256
NCB = SEQ // CB
NB = 4
NBG = 2 * NB
KW = (NB + 1) * CH
M = NB * CH
CPB = TOT // CH
PACK = 128
W = 128
NWORK = 32

NSPLIT = 2
HB = B // NSPLIT
GTOT_S = HB * TOT
NW_S = GTOT_S // W
NCHUNK_S = GTOT_S // CH
NRND_S = HB * NH
SELF_MASK_VAL = -50000.0
NEG_BIG = -1e30


def _hash_sort_body(qk_ref, rot_ref, slot_ref):
    bh = pl.program_id(0)
    qk = qk_ref[0]
    rot = rot_ref[0]
    rvT = lax.dot_general(rot.astype(jnp.bfloat16), qk.astype(jnp.bfloat16),
                          (((0,), (1,)), ((), ())),
                          preferred_element_type=jnp.float32)
    m1 = jnp.max(rvT, axis=0, keepdims=True)
    m2 = jnp.max(-rvT, axis=0, keepdims=True)
    sub = lax.broadcasted_iota(jnp.int32, (NROT, SEQ), 0)
    idx1 = jnp.min(jnp.where(rvT == m1, sub, NBKT), axis=0, keepdims=True)
    idx2 = jnp.min(jnp.where(-rvT == m2, sub, NBKT), axis=0, keepdims=True)
    bucket = jnp.where(m1 >= m2, idx1, idx2 + NROT)

    bsub = lax.broadcasted_iota(jnp.int32, (NBKT, SEQ), 0)
    onehot = bsub == bucket
    oh_f = onehot.astype(jnp.float32)
    oh_b = onehot.astype(jnp.bfloat16)

    hist = jnp.sum(oh_f, axis=1, keepdims=True)
    inc = hist
    for sh in (1, 2, 4, 8, 16, 32):
        inc = inc + jnp.concatenate(
            [jnp.zeros((sh, 1), jnp.float32), inc[:-sh, :]], axis=0)
    start = inc - hist

    ri = lax.broadcasted_iota(jnp.int32, (CB, CB), 0)
    ci = lax.broadcasted_iota(jnp.int32, (CB, CB), 1)
    utri = (ri < ci).astype(jnp.bfloat16)

    base = jnp.zeros((NBKT, 1), jnp.float32)
    for k in range(NCB):
        ohk_b = oh_b[:, k * CB:(k + 1) * CB]
        ohk_f = oh_f[:, k * CB:(k + 1) * CB]
        cumk = lax.dot_general(ohk_b, utri, (((1,), (0,)), ((), ())),
                               preferred_element_type=jnp.float32)
        pos = jnp.sum(ohk_f * (cumk + base + start), axis=0, keepdims=True)
        slot_ref[0, :, k * CB:(k + 1) * CB] = (
            pos.astype(jnp.int32) + (bh % NRND_S) * SEQ)
        base = base + jnp.sum(ohk_f, axis=1, keepdims=True)


def _hash_sort(qk, rot_h):
    return pl.pallas_call(
        _hash_sort_body,
        grid=(B * NH,),
        in_specs=[
            pl.BlockSpec((1, SEQ, DIM), lambda bh: (bh // NH, 0, 0)),
            pl.BlockSpec((1, DIM, NROT), lambda bh: (bh % NH, 0, 0)),
        ],
        out_specs=pl.BlockSpec((1, 1, SEQ), lambda bh: (bh, 0, 0)),
        out_shape=jax.ShapeDtypeStruct((B * NH, 1, SEQ), jnp.int32),
    )(qk, rot_h)


@functools.lru_cache(maxsize=None)
def _sc_mesh():
    return plsc.VectorSubcoreMesh(core_axis_name="c", subcore_axis_name="s")


def _sc_compiler_params():
    cp = pltpu.CompilerParams()
    if "needs_layout_passes" in pltpu.CompilerParams.__dataclass_fields__:
        cp = dataclasses.replace(cp, needs_layout_passes=False)
    return cp


@functools.lru_cache(maxsize=None)
def _sc_scatter():
    @functools.partial(
        pl.kernel,
        compiler_params=_sc_compiler_params(),
        out_type=[
            jax.ShapeDtypeStruct((GTOT_S, 2 * DIM), jnp.float32),
            jax.ShapeDtypeStruct((GTOT_S,), jnp.int32),
        ],
        mesh=_sc_mesh(),
        scratch_types=[
            pltpu.VMEM((4, W), jnp.int32),
            pltpu.VMEM((4, W, 2 * DIM), jnp.float32),
            pltpu.VMEM((SEQ,), jnp.int32),
            pltpu.VMEM((SEQ,), jnp.int32),
            pltpu.SemaphoreType.DMA,
            pltpu.SemaphoreType.DMA,
        ],
    )
    def body(qkv_hbm, slot_hbm, sqkv_hbm, st_hbm, idx_v, row_v, slot_v, st_v,
             sem_i, sem_o):
        wid = lax.axis_index("s") * 2 + lax.axis_index("c")
        wpb = TOT // W
        wpr = SEQ // W

        @pl.loop(0, NW_S // NWORK // 4)
        def _(g):
            loads = []
            for b_ in range(4):
                w = wid + (g * 4 + b_) * NWORK
                bb = w // wpb
                t0 = (w % wpr) * W
                row0 = bb * SEQ + t0
                loads.append(pltpu.async_copy(
                    slot_hbm.at[pl.ds(w * W, W)], idx_v.at[b_], sem_i))
                loads.append(pltpu.async_copy(
                    qkv_hbm.at[pl.ds(row0, W)], row_v.at[b_], sem_i))
            for h in loads:
                h.wait()
            stores = [pltpu.async_copy(row_v.at[b_], sqkv_hbm.at[idx_v.at[b_]],
                                       sem_o)
                      for b_ in range(4)]
            for h in stores:
                h.wait()

        @pl.loop(0, NRND_S // NWORK)
        def _(r):
            bh = wid + r * NWORK
            pltpu.sync_copy(slot_hbm.at[pl.ds(bh * SEQ, SEQ)], slot_v)
            base = bh * SEQ

            @pl.loop(0, SEQ // 16)
            def _(k):
                sl = pl.ds(k * 16, 16)
                loc = slot_v[sl] - base
                tvec = lax.iota(jnp.int32, 16) + k * 16
                plsc.store_scatter(st_v, [loc], tvec)

            pltpu.sync_copy(st_v, st_hbm.at[pl.ds(base, SEQ)])

    return body


def _attn_body(qv_ref, qvh_ref, stq_ref, stk_ref, sth_ref, out_ref):
    for h_ in range(2):
        off = h_ * NB
        qv = qv_ref[off:off + NB].reshape(M, 2 * DIM)
        if h_ == 0:
            qvh = qvh_ref[...].reshape(CH, 2 * DIM)
            sth = sth_ref[...].reshape(1, CH)
        else:
            qvh = qv_ref[off - 1]
            sth = stk_ref[off - 1]
        q = qv[:, :DIM]
        kq = jnp.concatenate([qvh[:, :DIM], q], axis=0)
        vv = jnp.concatenate([qvh[:, DIM:], qv[:, DIM:]], axis=0)
        qt = stq_ref[off:off + NB].reshape(M, 1)
        stk = stk_ref[off:off + NB]

        nrm = jnp.sqrt(jnp.sum(kq * kq, axis=1, keepdims=True))
        bk = (kq / jnp.maximum(nrm, 1e-12)).astype(jnp.bfloat16)

        dots = lax.dot_general(q.astype(jnp.bfloat16), bk,
                               (((1,), (1,)), ((), ())),
                               preferred_element_type=jnp.float32)
        dots = dots * (DIM ** -0.5)
        rowi = lax.broadcasted_iota(jnp.int32, (M, 1), 0)
        pieces = []
        for kc in range(NB + 1):
            krow = (sth if kc == 0 else stk[kc - 1]).reshape(1, CH)
            d_c = jnp.where(qt == krow, SELF_MASK_VAL,
                            dots[:, kc * CH:(kc + 1) * CH])
            inband = (rowi >= (kc - 1) * CH) & (rowi < (kc + 1) * CH)
            pieces.append(jnp.where(inband, d_c, NEG_BIG))
        dots = jnp.concatenate(pieces, axis=1)

        mx = jnp.max(dots, axis=1, keepdims=True)
        p = jnp.exp(dots - mx)
        s = jnp.sum(p, axis=1, keepdims=True)
        lse = mx + jnp.log(s)
        probs = (p / s).astype(jnp.bfloat16)
        bo = lax.dot_general(probs, vv.astype(jnp.bfloat16),
                             (((1,), (0,)), ((), ())),
                             preferred_element_type=jnp.float32)
        out = jnp.concatenate(
            [bo, lse, jnp.zeros((M, PACK - DIM - 1), jnp.float32)], axis=1)
        out_ref[off:off + NB] = out.reshape(NB, CH, PACK)


def _halo_idx(c):
    g0 = c * NBG
    return jnp.where(g0 % CPB == 0, g0 + CPB - 1, g0 - 1)


def _attention(sqkv3, stq3, stk3):
    return pl.pallas_call(
        _attn_body,
        grid=(NCHUNK_S // NBG,),
        in_specs=[
            pl.BlockSpec((NBG, CH, 2 * DIM), lambda c: (c, 0, 0)),
            pl.BlockSpec((1, CH, 2 * DIM), lambda c: (_halo_idx(c), 0, 0)),
            pl.BlockSpec((NBG, CH, 1), lambda c: (c, 0, 0)),
            pl.BlockSpec((NBG, 1, CH), lambda c: (c, 0, 0)),
            pl.BlockSpec((1, 1, CH), lambda c: (_halo_idx(c), 0, 0)),
        ],
        out_specs=pl.BlockSpec((NBG, CH, PACK), lambda c: (c, 0, 0)),
        out_shape=jax.ShapeDtypeStruct((NCHUNK_S, CH, PACK), jnp.float32),
    )(sqkv3, sqkv3, stq3, stk3, stk3)


@functools.lru_cache(maxsize=None)
def _sc_unsort():
    @functools.partial(
        pl.kernel,
        out_type=jax.ShapeDtypeStruct((GTOT_S, PACK), jnp.float32),
        mesh=_sc_mesh(),
        scratch_types=[
            pltpu.VMEM((4, W), jnp.int32),
            pltpu.VMEM((4, W, PACK), jnp.float32),
            pltpu.SemaphoreType.DMA,
            pltpu.SemaphoreType.DMA,
        ],
    )
    def body(so_hbm, slot_hbm, out_hbm, idx_v, rows_v, sem_i, sem_o):
        wid = lax.axis_index("s") * 2 + lax.axis_index("c")

        @pl.loop(0, NW_S // NWORK // 4)
        def _(g):
            loads = [pltpu.async_copy(
                slot_hbm.at[pl.ds((wid + (g * 4 + b_) * NWORK) * W, W)],
                idx_v.at[b_], sem_i) for b_ in range(4)]
            for h in loads:
                h.wait()
            gathers = [pltpu.async_copy(so_hbm.at[idx_v.at[b_]],
                                        rows_v.at[b_], sem_i)
                       for b_ in range(4)]
            for h in gathers:
                h.wait()
            stores = [pltpu.async_copy(
                rows_v.at[b_],
                out_hbm.at[pl.ds((wid + (g * 4 + b_) * NWORK) * W, W)], sem_o)
                for b_ in range(4)]
            for h in stores:
                h.wait()

    return body


TB = 512


def _combine_body(g_ref, out_ref):
    g = g_ref[0]
    o = g[:, :, :DIM]
    lg = g[:, :, DIM:DIM + 1]
    mx = jnp.max(lg, axis=0, keepdims=True)
    wgt = jnp.exp(lg - mx)
    s = jnp.sum(wgt, axis=0, keepdims=True)
    wn = wgt / s
    out_ref[0] = jnp.sum(o * wn, axis=0)


def _combine(g4):
    return pl.pallas_call(
        _combine_body,
        grid=(HB, SEQ // TB),
        in_specs=[pl.BlockSpec((1, NH, TB, PACK), lambda b, t: (b, 0, t, 0))],
        out_specs=pl.BlockSpec((1, TB, DIM), lambda b, t: (b, t, 0)),
        out_shape=jax.ShapeDtypeStruct((HB, SEQ, DIM), jnp.float32),
    )(g4)


def kernel(qk, v, rotations):
    rot_h = jnp.transpose(rotations[0], (1, 0, 2))
    slot_all = _hash_sort(qk, rot_h)

    qkv = jnp.concatenate(
        [qk.reshape(B * SEQ, DIM), v.reshape(B * SEQ, DIM)], axis=1)

    outs = []
    for s in range(NSPLIT):
        slot_s = slot_all[s * NRND_S:(s + 1) * NRND_S].reshape(GTOT_S)
        qkv_s = qkv[s * HB * SEQ:(s + 1) * HB * SEQ]
        sqkv, st = _sc_scatter()(qkv_s, slot_s)
        so = _attention(sqkv.reshape(NCHUNK_S, CH, 2 * DIM),
                        st.reshape(NCHUNK_S, CH, 1),
                        st.reshape(NCHUNK_S, 1, CH))
        g = _sc_unsort()(so.reshape(GTOT_S, PACK), slot_s)
        outs.append(_combine(g.reshape(HB, NH, SEQ, PACK)))
    return jnp.concatenate(outs, axis=0)


# speedup vs baseline: 1.4129x; 1.0740x over previous
import dataclasses
import functools

import jax
import jax.numpy as jnp
from jax import lax
from jax.experimental import pallas as pl
from jax.experimental.pallas import tpu as pltpu
from jax.experimental.pallas import tpu_sc as plsc

B = 8
SEQ = 4096
DIM = 64
NH = 8
NBKT = 64
NROT = NBKT // 2
CH = 64
TOT = NH * SEQ
GTOT = B * TOT
CB = 256
NCB = SEQ // CB
NB = 4
NBG = 4 * NB
KW = (NB + 1) * CH
M = NB * CH
CPB = TOT // CH
PACK = 128
W = 128
NWORK = 32

NSPLIT = 2
HB = B // NSPLIT
GTOT_S = HB * TOT
NW_S = GTOT_S // W
NCHUNK_S = GTOT_S // CH
NRND_S = HB * NH
SELF_MASK_VAL = -50000.0
NEG_BIG = -1e30


def _hash_sort_body(qk_ref, rot_ref, slot_ref):
    bh = pl.program_id(0)
    qk = qk_ref[0]
    rot = rot_ref[0]
    rvT = lax.dot_general(rot.astype(jnp.bfloat16), qk.astype(jnp.bfloat16),
                          (((0,), (1,)), ((), ())),
                          preferred_element_type=jnp.float32)
    m1 = jnp.max(rvT, axis=0, keepdims=True)
    m2 = jnp.max(-rvT, axis=0, keepdims=True)
    sub = lax.broadcasted_iota(jnp.int32, (NROT, SEQ), 0)
    idx1 = jnp.min(jnp.where(rvT == m1, sub, NBKT), axis=0, keepdims=True)
    idx2 = jnp.min(jnp.where(-rvT == m2, sub, NBKT), axis=0, keepdims=True)
    bucket = jnp.where(m1 >= m2, idx1, idx2 + NROT)

    bsub = lax.broadcasted_iota(jnp.int32, (NBKT, SEQ), 0)
    onehot = bsub == bucket
    oh_f = onehot.astype(jnp.float32)
    oh_b = onehot.astype(jnp.bfloat16)

    hist = jnp.sum(oh_f, axis=1, keepdims=True)
    inc = hist
    for sh in (1, 2, 4, 8, 16, 32):
        inc = inc + jnp.concatenate(
            [jnp.zeros((sh, 1), jnp.float32), inc[:-sh, :]], axis=0)
    start = inc - hist

    ri = lax.broadcasted_iota(jnp.int32, (CB, CB), 0)
    ci = lax.broadcasted_iota(jnp.int32, (CB, CB), 1)
    utri = (ri < ci).astype(jnp.bfloat16)

    base = jnp.zeros((NBKT, 1), jnp.float32)
    for k in range(NCB):
        ohk_b = oh_b[:, k * CB:(k + 1) * CB]
        ohk_f = oh_f[:, k * CB:(k + 1) * CB]
        cumk = lax.dot_general(ohk_b, utri, (((1,), (0,)), ((), ())),
                               preferred_element_type=jnp.float32)
        pos = jnp.sum(ohk_f * (cumk + base + start), axis=0, keepdims=True)
        slot_ref[0, :, k * CB:(k + 1) * CB] = (
            pos.astype(jnp.int32) + (bh % NRND_S) * SEQ)
        base = base + jnp.sum(ohk_f, axis=1, keepdims=True)


def _hash_sort(qk, rot_h):
    return pl.pallas_call(
        _hash_sort_body,
        grid=(B * NH,),
        in_specs=[
            pl.BlockSpec((1, SEQ, DIM), lambda bh: (bh // NH, 0, 0)),
            pl.BlockSpec((1, DIM, NROT), lambda bh: (bh % NH, 0, 0)),
        ],
        out_specs=pl.BlockSpec((1, 1, SEQ), lambda bh: (bh, 0, 0)),
        out_shape=jax.ShapeDtypeStruct((B * NH, 1, SEQ), jnp.int32),
    )(qk, rot_h)


@functools.lru_cache(maxsize=None)
def _sc_mesh():
    return plsc.VectorSubcoreMesh(core_axis_name="c", subcore_axis_name="s")


def _sc_compiler_params():
    cp = pltpu.CompilerParams()
    if "needs_layout_passes" in pltpu.CompilerParams.__dataclass_fields__:
        cp = dataclasses.replace(cp, needs_layout_passes=False)
    return cp


@functools.lru_cache(maxsize=None)
def _sc_scatter():
    @functools.partial(
        pl.kernel,
        compiler_params=_sc_compiler_params(),
        out_type=[
            jax.ShapeDtypeStruct((GTOT_S, 2 * DIM), jnp.float32),
            jax.ShapeDtypeStruct((GTOT_S,), jnp.int32),
        ],
        mesh=_sc_mesh(),
        scratch_types=[
            pltpu.VMEM((4, W), jnp.int32),
            pltpu.VMEM((4, W, 2 * DIM), jnp.float32),
            pltpu.VMEM((SEQ,), jnp.int32),
            pltpu.VMEM((SEQ,), jnp.int32),
            pltpu.SemaphoreType.DMA,
            pltpu.SemaphoreType.DMA,
        ],
    )
    def body(qkv_hbm, slot_hbm, sqkv_hbm, st_hbm, idx_v, row_v, slot_v, st_v,
             sem_i, sem_o):
        wid = lax.axis_index("s") * 2 + lax.axis_index("c")
        wpb = TOT // W
        wpr = SEQ // W

        @pl.loop(0, NW_S // NWORK // 4)
        def _(g):
            loads = []
            for b_ in range(4):
                w = wid + (g * 4 + b_) * NWORK
                bb = w // wpb
                t0 = (w % wpr) * W
                row0 = bb * SEQ + t0
                loads.append(pltpu.async_copy(
                    slot_hbm.at[pl.ds(w * W, W)], idx_v.at[b_], sem_i))
                loads.append(pltpu.async_copy(
                    qkv_hbm.at[pl.ds(row0, W)], row_v.at[b_], sem_i))
            for h in loads:
                h.wait()
            stores = [pltpu.async_copy(row_v.at[b_], sqkv_hbm.at[idx_v.at[b_]],
                                       sem_o)
                      for b_ in range(4)]
            for h in stores:
                h.wait()

        @pl.loop(0, NRND_S // NWORK)
        def _(r):
            bh = wid + r * NWORK
            pltpu.sync_copy(slot_hbm.at[pl.ds(bh * SEQ, SEQ)], slot_v)
            base = bh * SEQ

            @pl.loop(0, SEQ // 16)
            def _(k):
                sl = pl.ds(k * 16, 16)
                loc = slot_v[sl] - base
                tvec = lax.iota(jnp.int32, 16) + k * 16
                plsc.store_scatter(st_v, [loc], tvec)

            pltpu.sync_copy(st_v, st_hbm.at[pl.ds(base, SEQ)])

    return body


def _attn_body(qv_ref, qvh_ref, stq_ref, stk_ref, sth_ref, out_ref):
    for h_ in range(NBG // NB):
        off = h_ * NB
        qv = qv_ref[off:off + NB].reshape(M, 2 * DIM)
        if h_ == 0:
            qvh = qvh_ref[...].reshape(CH, 2 * DIM)
            sth = sth_ref[...].reshape(1, CH)
        else:
            qvh = qv_ref[off - 1]
            sth = stk_ref[off - 1]
        q = qv[:, :DIM]
        kq = jnp.concatenate([qvh[:, :DIM], q], axis=0)
        vv = jnp.concatenate([qvh[:, DIM:], qv[:, DIM:]], axis=0)
        qt = stq_ref[off:off + NB].reshape(M, 1)
        stk = stk_ref[off:off + NB]

        nrm = jnp.sqrt(jnp.sum(kq * kq, axis=1, keepdims=True))
        bk = (kq / jnp.maximum(nrm, 1e-12)).astype(jnp.bfloat16)

        dots = lax.dot_general(q.astype(jnp.bfloat16), bk,
                               (((1,), (1,)), ((), ())),
                               preferred_element_type=jnp.float32)
        dots = dots * (DIM ** -0.5)
        rowi = lax.broadcasted_iota(jnp.int32, (M, 1), 0)
        pieces = []
        for kc in range(NB + 1):
            krow = (sth if kc == 0 else stk[kc - 1]).reshape(1, CH)
            d_c = jnp.where(qt == krow, SELF_MASK_VAL,
                            dots[:, kc * CH:(kc + 1) * CH])
            inband = (rowi >= (kc - 1) * CH) & (rowi < (kc + 1) * CH)
            pieces.append(jnp.where(inband, d_c, NEG_BIG))
        dots = jnp.concatenate(pieces, axis=1)

        mx = jnp.max(dots, axis=1, keepdims=True)
        p = jnp.exp(dots - mx)
        s = jnp.sum(p, axis=1, keepdims=True)
        lse = mx + jnp.log(s)
        probs = (p / s).astype(jnp.bfloat16)
        bo = lax.dot_general(probs, vv.astype(jnp.bfloat16),
                             (((1,), (0,)), ((), ())),
                             preferred_element_type=jnp.float32)
        out = jnp.concatenate(
            [bo, lse, jnp.zeros((M, PACK - DIM - 1), jnp.float32)], axis=1)
        out_ref[off:off + NB] = out.reshape(NB, CH, PACK)


def _halo_idx(c):
    g0 = c * NBG
    return jnp.where(g0 % CPB == 0, g0 + CPB - 1, g0 - 1)


def _attention(sqkv3, stq3, stk3):
    return pl.pallas_call(
        _attn_body,
        grid=(NCHUNK_S // NBG,),
        in_specs=[
            pl.BlockSpec((NBG, CH, 2 * DIM), lambda c: (c, 0, 0)),
            pl.BlockSpec((1, CH, 2 * DIM), lambda c: (_halo_idx(c), 0, 0)),
            pl.BlockSpec((NBG, CH, 1), lambda c: (c, 0, 0)),
            pl.BlockSpec((NBG, 1, CH), lambda c: (c, 0, 0)),
            pl.BlockSpec((1, 1, CH), lambda c: (_halo_idx(c), 0, 0)),
        ],
        out_specs=pl.BlockSpec((NBG, CH, PACK), lambda c: (c, 0, 0)),
        out_shape=jax.ShapeDtypeStruct((NCHUNK_S, CH, PACK), jnp.float32),
    )(sqkv3, sqkv3, stq3, stk3, stk3)


@functools.lru_cache(maxsize=None)
def _sc_unsort():
    @functools.partial(
        pl.kernel,
        out_type=jax.ShapeDtypeStruct((GTOT_S, PACK), jnp.float32),
        mesh=_sc_mesh(),
        scratch_types=[
            pltpu.VMEM((4, W), jnp.int32),
            pltpu.VMEM((4, W, PACK), jnp.float32),
            pltpu.SemaphoreType.DMA,
            pltpu.SemaphoreType.DMA,
        ],
    )
    def body(so_hbm, slot_hbm, out_hbm, idx_v, rows_v, sem_i, sem_o):
        wid = lax.axis_index("s") * 2 + lax.axis_index("c")

        @pl.loop(0, NW_S // NWORK // 4)
        def _(g):
            loads = [pltpu.async_copy(
                slot_hbm.at[pl.ds((wid + (g * 4 + b_) * NWORK) * W, W)],
                idx_v.at[b_], sem_i) for b_ in range(4)]
            for h in loads:
                h.wait()
            gathers = [pltpu.async_copy(so_hbm.at[idx_v.at[b_]],
                                        rows_v.at[b_], sem_i)
                       for b_ in range(4)]
            for h in gathers:
                h.wait()
            stores = [pltpu.async_copy(
                rows_v.at[b_],
                out_hbm.at[pl.ds((wid + (g * 4 + b_) * NWORK) * W, W)], sem_o)
                for b_ in range(4)]
            for h in stores:
                h.wait()

    return body


TB = 512


def _combine_body(g_ref, out_ref):
    g = g_ref[0]
    o = g[:, :, :DIM]
    lg = g[:, :, DIM:DIM + 1]
    mx = jnp.max(lg, axis=0, keepdims=True)
    wgt = jnp.exp(lg - mx)
    s = jnp.sum(wgt, axis=0, keepdims=True)
    wn = wgt / s
    out_ref[0] = jnp.sum(o * wn, axis=0)


def _combine(g4):
    return pl.pallas_call(
        _combine_body,
        grid=(HB, SEQ // TB),
        in_specs=[pl.BlockSpec((1, NH, TB, PACK), lambda b, t: (b, 0, t, 0))],
        out_specs=pl.BlockSpec((1, TB, DIM), lambda b, t: (b, t, 0)),
        out_shape=jax.ShapeDtypeStruct((HB, SEQ, DIM), jnp.float32),
    )(g4)


def kernel(qk, v, rotations):
    rot_h = jnp.transpose(rotations[0], (1, 0, 2))
    slot_all = _hash_sort(qk, rot_h)

    qkv = jnp.concatenate(
        [qk.reshape(B * SEQ, DIM), v.reshape(B * SEQ, DIM)], axis=1)

    outs = []
    for s in range(NSPLIT):
        slot_s = slot_all[s * NRND_S:(s + 1) * NRND_S].reshape(GTOT_S)
        qkv_s = qkv[s * HB * SEQ:(s + 1) * HB * SEQ]
        sqkv, st = _sc_scatter()(qkv_s, slot_s)
        so = _attention(sqkv.reshape(NCHUNK_S, CH, 2 * DIM),
                        st.reshape(NCHUNK_S, CH, 1),
                        st.reshape(NCHUNK_S, 1, CH))
        g = _sc_unsort()(so.reshape(GTOT_S, PACK), slot_s)
        outs.append(_combine(g.reshape(HB, NH, SEQ, PACK)))
    return jnp.concatenate(outs, axis=0)
